# Initial kernel scaffold; baseline (speedup 1.0000x reference)
import jax
import jax.numpy as jnp
from jax.experimental import pallas as pl


def kernel(x, edge_index, W1, b1, W2, b2):
    raise NotImplementedError("write your pallas kernel here")


# speedup vs baseline: 95.0825x; 95.0825x over previous
import functools

import jax
import jax.numpy as jnp
from jax import lax
from jax.experimental import pallas as pl
from jax.experimental.pallas import tpu as pltpu
from jax.experimental.pallas import tpu_sc as plsc

_N = 100000
_NP = 100096
_E = 6400000
_EPAD = 6553600
_R = _EPAD // 128
_NC = 2
_NS = 16
_NW = _NC * _NS
_RT = _R // _NW
_NPT = _NP // _NS

_KD = 16
_KA = 8

_f32 = jnp.float32
_mesh = plsc.VectorSubcoreMesh(core_axis_name="c", subcore_axis_name="s",
                               num_cores=_NC, num_subcores=_NS)


@functools.partial(
    pl.kernel,
    out_type=jax.ShapeDtypeStruct((_NC * _NP,), _f32),
    mesh=_mesh,
    compiler_params=pltpu.CompilerParams(use_tc_tiling_on_sc=False),
    scratch_types=[
        pltpu.VMEM_SHARED((_NP,), _f32),
        pltpu.VMEM((_NPT,), _f32),
        pltpu.VMEM((_KD, 128), jnp.int32),
        pltpu.SemaphoreType.DMA,
    ],
)
def _sc_degree(dst_hbm, out_hbm, acc_sh, ones_v, dst_v, sem):
    c = lax.axis_index("c")
    s = lax.axis_index("s")
    wid = c * _NS + s

    @pl.loop(0, _NPT // 16)
    def _fill(i):
        ones_v[pl.ds(i * 16, 16)] = jnp.ones((16,), _f32)

    nb = s * _NPT
    pltpu.sync_copy(ones_v, acc_sh.at[pl.ds(nb, _NPT)])
    plsc.subcore_barrier()

    row0 = wid * _RT

    @pl.loop(0, _RT // _KD)
    def _chunk(ch):
        pltpu.sync_copy(dst_hbm.at[pl.ds(row0 + ch * _KD, _KD)], dst_v)
        descs = [
            pltpu.async_copy(ones_v.at[pl.ds(0, 128)],
                             acc_sh.at[dst_v.at[j]], sem, add=True)
            for j in range(_KD)
        ]
        for d in descs:
            d.wait()

    plsc.subcore_barrier()
    pltpu.sync_copy(acc_sh.at[pl.ds(nb, _NPT)], ones_v)
    pltpu.sync_copy(ones_v, out_hbm.at[pl.ds(c * _NP + nb, _NPT)])


@functools.partial(
    pl.kernel,
    out_type=(jax.ShapeDtypeStruct((_NC * _NP,), _f32),
              jax.ShapeDtypeStruct((_NC * _NP,), _f32)),
    mesh=_mesh,
    compiler_params=pltpu.CompilerParams(use_tc_tiling_on_sc=False),
    scratch_types=[
        pltpu.VMEM_SHARED((_NP,), _f32),
        pltpu.VMEM_SHARED((_NP,), _f32),
        pltpu.VMEM_SHARED((_NP,), _f32),
        pltpu.VMEM_SHARED((_NP,), _f32),
        pltpu.VMEM((_NPT,), _f32),
        pltpu.VMEM((_KA, 128), jnp.int32),
        pltpu.VMEM((_KA, 128), jnp.int32),
        pltpu.VMEM((_KA, 128), _f32),
        pltpu.VMEM((_KA, 128), _f32),
        pltpu.SemaphoreType.DMA,
        pltpu.SemaphoreType.DMA,
    ],
)
def _sc_aggregate(y0_hbm, y1_hbm, src_hbm, dst_hbm, o0_hbm, o1_hbm,
                  t0_sh, t1_sh, a0_sh, a1_sh, stage_v, src_v, dst_v,
                  m0_v, m1_v, gsem, ssem):
    c = lax.axis_index("c")
    s = lax.axis_index("s")
    wid = c * _NS + s
    nb = s * _NPT

    pltpu.sync_copy(y0_hbm.at[pl.ds(nb, _NPT)], stage_v)
    pltpu.sync_copy(stage_v, t0_sh.at[pl.ds(nb, _NPT)])
    pltpu.sync_copy(stage_v, a0_sh.at[pl.ds(nb, _NPT)])
    pltpu.sync_copy(y1_hbm.at[pl.ds(nb, _NPT)], stage_v)
    pltpu.sync_copy(stage_v, t1_sh.at[pl.ds(nb, _NPT)])
    pltpu.sync_copy(stage_v, a1_sh.at[pl.ds(nb, _NPT)])
    plsc.subcore_barrier()

    row0 = wid * _RT

    @pl.loop(0, _RT // _KA)
    def _chunk(ch):
        rbase = row0 + ch * _KA
        pltpu.sync_copy(src_hbm.at[pl.ds(rbase, _KA)], src_v)
        pltpu.sync_copy(dst_hbm.at[pl.ds(rbase, _KA)], dst_v)
        gd = [
            pltpu.async_copy(t0_sh.at[src_v.at[j]], m0_v.at[j], gsem)
            for j in range(_KA)
        ] + [
            pltpu.async_copy(t1_sh.at[src_v.at[j]], m1_v.at[j], gsem)
            for j in range(_KA)
        ]
        for d in gd:
            d.wait()
        sd = [
            pltpu.async_copy(m0_v.at[j], a0_sh.at[dst_v.at[j]], ssem,
                             add=True)
            for j in range(_KA)
        ] + [
            pltpu.async_copy(m1_v.at[j], a1_sh.at[dst_v.at[j]], ssem,
                             add=True)
            for j in range(_KA)
        ]
        for d in sd:
            d.wait()

    plsc.subcore_barrier()
    pltpu.sync_copy(a0_sh.at[pl.ds(nb, _NPT)], stage_v)
    pltpu.sync_copy(stage_v, o0_hbm.at[pl.ds(c * _NP + nb, _NPT)])
    pltpu.sync_copy(a1_sh.at[pl.ds(nb, _NPT)], stage_v)
    pltpu.sync_copy(stage_v, o1_hbm.at[pl.ds(c * _NP + nb, _NPT)])


def _stage_norm(degp, xt):
    def body(degp_ref, xt_ref, yst_ref, dis_ref):
        deg = degp_ref[0:1, :] + degp_ref[1:2, :] - 1.0
        dis = lax.rsqrt(deg)
        dis_ref[...] = dis
        yst_ref[...] = xt_ref[...] * dis

    return pl.pallas_call(
        body,
        out_shape=(
            jax.ShapeDtypeStruct((2, _NP), _f32),
            jax.ShapeDtypeStruct((1, _NP), _f32),
        ),
    )(degp, xt)


def _stage_layer1(a4, yst, dis, W1, b1):
    def body(a_ref, yst_ref, dis_ref, w_ref, b_ref, out_ref):
        a = a_ref[...]
        ys = yst_ref[...]
        dis = dis_ref[...]
        z0 = (a[0:1] + a[1:2] - ys[0:1]) * dis
        z1 = (a[2:3] + a[3:4] - ys[1:2]) * dis
        h0 = jnp.maximum(z0 * w_ref[0, 0] + z1 * w_ref[1, 0] + b_ref[0], 0.0)
        h1 = jnp.maximum(z0 * w_ref[0, 1] + z1 * w_ref[1, 1] + b_ref[1], 0.0)
        out_ref[0:1, :] = h0 * dis
        out_ref[1:2, :] = h1 * dis

    return pl.pallas_call(
        body,
        in_specs=[
            pl.BlockSpec(memory_space=pltpu.VMEM),
            pl.BlockSpec(memory_space=pltpu.VMEM),
            pl.BlockSpec(memory_space=pltpu.VMEM),
            pl.BlockSpec(memory_space=pltpu.SMEM),
            pl.BlockSpec(memory_space=pltpu.SMEM),
        ],
        out_shape=jax.ShapeDtypeStruct((2, _NP), _f32),
    )(a4, yst, dis, W1, b1)


def _stage_layer2(a4, yst2, dis, W2, b2):
    def body(a_ref, yst_ref, dis_ref, w_ref, b_ref, out_ref):
        a = a_ref[...]
        ys = yst_ref[...]
        dis = dis_ref[...]
        z0 = (a[0:1] + a[1:2] - ys[0:1]) * dis
        z1 = (a[2:3] + a[3:4] - ys[1:2]) * dis
        out_ref[...] = z0 * w_ref[0, 0] + z1 * w_ref[1, 0] + b_ref[0]

    return pl.pallas_call(
        body,
        in_specs=[
            pl.BlockSpec(memory_space=pltpu.VMEM),
            pl.BlockSpec(memory_space=pltpu.VMEM),
            pl.BlockSpec(memory_space=pltpu.VMEM),
            pl.BlockSpec(memory_space=pltpu.SMEM),
            pl.BlockSpec(memory_space=pltpu.SMEM),
        ],
        out_shape=jax.ShapeDtypeStruct((1, _NP), _f32),
    )(a4, yst2, dis, W2, b2)


def kernel(x, edge_index, W1, b1, W2, b2):
    pad_idx = jnp.full((_EPAD - _E,), _NP - 1, jnp.int32)
    src = jnp.concatenate([edge_index[0].astype(jnp.int32), pad_idx])
    dst = jnp.concatenate([edge_index[1].astype(jnp.int32), pad_idx])
    src = src.reshape(_R, 128)
    dst = dst.reshape(_R, 128)
    xt = jnp.pad(x, ((0, _NP - _N), (0, 0))).T

    degp = _sc_degree(dst).reshape(_NC, _NP)
    yst, dis = _stage_norm(degp, xt)

    o0, o1 = _sc_aggregate(yst[0], yst[1], src, dst)
    a4 = jnp.concatenate([o0.reshape(_NC, _NP), o1.reshape(_NC, _NP)])
    yst2 = _stage_layer1(a4, yst, dis, W1, b1)

    o0, o1 = _sc_aggregate(yst2[0], yst2[1], src, dst)
    a4 = jnp.concatenate([o0.reshape(_NC, _NP), o1.reshape(_NC, _NP)])
    orow = _stage_layer2(a4, yst2, dis, W2, b2)

    return orow[0, :_N].reshape(_N, 1)


# speedup vs baseline: 132.9546x; 1.3983x over previous
import functools

import jax
import jax.numpy as jnp
from jax import lax
from jax.experimental import pallas as pl
from jax.experimental.pallas import tpu as pltpu
from jax.experimental.pallas import tpu_sc as plsc

_N = 100000
_NP = 100096
_E = 6400000
_EPAD = 6553600
_NC = 2
_NS = 16
_NW = _NC * _NS
_ET = _EPAD // _NW
_CH = 4096
_NCH = _ET // _CH
_NPT = _NP // _NS

_f32 = jnp.float32
_mesh = plsc.VectorSubcoreMesh(core_axis_name="c", subcore_axis_name="s",
                               num_cores=_NC, num_subcores=_NS)
_sc_params = pltpu.CompilerParams(use_tc_tiling_on_sc=False)


@functools.partial(
    pl.kernel,
    out_type=jax.ShapeDtypeStruct((_NC * _NP,), _f32),
    mesh=_mesh,
    compiler_params=_sc_params,
    scratch_types=[
        pltpu.VMEM_SHARED((_NP,), _f32),
        pltpu.VMEM((_NPT,), _f32),
        pltpu.VMEM((_CH,), jnp.int32),
        pltpu.SemaphoreType.DMA,
    ],
)
def _sc_degree(dst_hbm, out_hbm, acc_sh, ones_v, dst_v, sem):
    c = lax.axis_index("c")
    s = lax.axis_index("s")
    wid = c * _NS + s

    @pl.loop(0, _NPT // 16)
    def _fill(i):
        ones_v[pl.ds(i * 16, 16)] = jnp.ones((16,), _f32)

    nb = s * _NPT
    pltpu.sync_copy(ones_v, acc_sh.at[pl.ds(nb, _NPT)])
    plsc.subcore_barrier()

    e0 = wid * _ET

    @pl.loop(0, _NCH)
    def _chunk(ch):
        pltpu.sync_copy(dst_hbm.at[pl.ds(e0 + ch * _CH, _CH)], dst_v)
        pltpu.async_copy(ones_v.at[pl.ds(0, _CH)], acc_sh.at[dst_v], sem,
                         add=True).wait()

    plsc.subcore_barrier()
    pltpu.sync_copy(acc_sh.at[pl.ds(nb, _NPT)], ones_v)
    pltpu.sync_copy(ones_v, out_hbm.at[pl.ds(c * _NP + nb, _NPT)])


@functools.partial(
    pl.kernel,
    out_type=(jax.ShapeDtypeStruct((_NC * _NP,), _f32),
              jax.ShapeDtypeStruct((_NC * _NP,), _f32)),
    mesh=_mesh,
    compiler_params=_sc_params,
    scratch_types=[
        pltpu.VMEM_SHARED((_NP,), _f32),
        pltpu.VMEM_SHARED((_NP,), _f32),
        pltpu.VMEM_SHARED((_NP,), _f32),
        pltpu.VMEM_SHARED((_NP,), _f32),
        pltpu.VMEM((_NPT,), _f32),
        pltpu.VMEM((_CH,), jnp.int32),
        pltpu.VMEM((_CH,), jnp.int32),
        pltpu.VMEM((_CH,), _f32),
        pltpu.VMEM((_CH,), _f32),
        pltpu.SemaphoreType.DMA,
        pltpu.SemaphoreType.DMA,
    ],
)
def _sc_aggregate2(y0_hbm, y1_hbm, src_hbm, dst_hbm, o0_hbm, o1_hbm,
                   t0_sh, t1_sh, a0_sh, a1_sh, stage_v, src_v, dst_v,
                   m0_v, m1_v, gsem, ssem):
    c = lax.axis_index("c")
    s = lax.axis_index("s")
    wid = c * _NS + s
    nb = s * _NPT

    pltpu.sync_copy(y0_hbm.at[pl.ds(nb, _NPT)], stage_v)
    pltpu.sync_copy(stage_v, t0_sh.at[pl.ds(nb, _NPT)])
    pltpu.sync_copy(stage_v, a0_sh.at[pl.ds(nb, _NPT)])
    pltpu.sync_copy(y1_hbm.at[pl.ds(nb, _NPT)], stage_v)
    pltpu.sync_copy(stage_v, t1_sh.at[pl.ds(nb, _NPT)])
    pltpu.sync_copy(stage_v, a1_sh.at[pl.ds(nb, _NPT)])
    plsc.subcore_barrier()

    e0 = wid * _ET

    @pl.loop(0, _NCH)
    def _chunk(ch):
        eb = e0 + ch * _CH
        pltpu.sync_copy(src_hbm.at[pl.ds(eb, _CH)], src_v)
        pltpu.sync_copy(dst_hbm.at[pl.ds(eb, _CH)], dst_v)
        g0 = pltpu.async_copy(t0_sh.at[src_v], m0_v, gsem)
        g1 = pltpu.async_copy(t1_sh.at[src_v], m1_v, gsem)
        g0.wait()
        g1.wait()
        s0 = pltpu.async_copy(m0_v, a0_sh.at[dst_v], ssem, add=True)
        s1 = pltpu.async_copy(m1_v, a1_sh.at[dst_v], ssem, add=True)
        s0.wait()
        s1.wait()

    plsc.subcore_barrier()
    pltpu.sync_copy(a0_sh.at[pl.ds(nb, _NPT)], stage_v)
    pltpu.sync_copy(stage_v, o0_hbm.at[pl.ds(c * _NP + nb, _NPT)])
    pltpu.sync_copy(a1_sh.at[pl.ds(nb, _NPT)], stage_v)
    pltpu.sync_copy(stage_v, o1_hbm.at[pl.ds(c * _NP + nb, _NPT)])


@functools.partial(
    pl.kernel,
    out_type=jax.ShapeDtypeStruct((_NC * _NP,), _f32),
    mesh=_mesh,
    compiler_params=_sc_params,
    scratch_types=[
        pltpu.VMEM_SHARED((_NP,), _f32),
        pltpu.VMEM_SHARED((_NP,), _f32),
        pltpu.VMEM((_NPT,), _f32),
        pltpu.VMEM((_CH,), jnp.int32),
        pltpu.VMEM((_CH,), jnp.int32),
        pltpu.VMEM((_CH,), _f32),
        pltpu.SemaphoreType.DMA,
        pltpu.SemaphoreType.DMA,
    ],
)
def _sc_aggregate1(u_hbm, src_hbm, dst_hbm, o_hbm,
                   t_sh, a_sh, stage_v, src_v, dst_v, m_v, gsem, ssem):
    c = lax.axis_index("c")
    s = lax.axis_index("s")
    wid = c * _NS + s
    nb = s * _NPT

    pltpu.sync_copy(u_hbm.at[pl.ds(nb, _NPT)], stage_v)
    pltpu.sync_copy(stage_v, t_sh.at[pl.ds(nb, _NPT)])
    pltpu.sync_copy(stage_v, a_sh.at[pl.ds(nb, _NPT)])
    plsc.subcore_barrier()

    e0 = wid * _ET

    @pl.loop(0, _NCH)
    def _chunk(ch):
        eb = e0 + ch * _CH
        pltpu.sync_copy(src_hbm.at[pl.ds(eb, _CH)], src_v)
        pltpu.sync_copy(dst_hbm.at[pl.ds(eb, _CH)], dst_v)
        pltpu.async_copy(t_sh.at[src_v], m_v, gsem).wait()
        pltpu.async_copy(m_v, a_sh.at[dst_v], ssem, add=True).wait()

    plsc.subcore_barrier()
    pltpu.sync_copy(a_sh.at[pl.ds(nb, _NPT)], stage_v)
    pltpu.sync_copy(stage_v, o_hbm.at[pl.ds(c * _NP + nb, _NPT)])


def _stage_norm(degp, xt):
    def body(degp_ref, xt_ref, yst_ref, dis_ref):
        deg = degp_ref[0:1, :] + degp_ref[1:2, :] - 1.0
        dis = lax.rsqrt(deg)
        dis_ref[...] = dis
        yst_ref[...] = xt_ref[...] * dis

    return pl.pallas_call(
        body,
        out_shape=(
            jax.ShapeDtypeStruct((2, _NP), _f32),
            jax.ShapeDtypeStruct((1, _NP), _f32),
        ),
    )(degp, xt)


def _stage_layer1(a4, yst, dis, W1, b1, W2):
    def body(a_ref, yst_ref, dis_ref, w1_ref, b1_ref, w2_ref, u_ref):
        a = a_ref[...]
        ys = yst_ref[...]
        dis = dis_ref[...]
        z0 = (a[0:1] + a[1:2] - ys[0:1]) * dis
        z1 = (a[2:3] + a[3:4] - ys[1:2]) * dis
        h0 = jnp.maximum(z0 * w1_ref[0, 0] + z1 * w1_ref[1, 0] + b1_ref[0],
                         0.0)
        h1 = jnp.maximum(z0 * w1_ref[0, 1] + z1 * w1_ref[1, 1] + b1_ref[1],
                         0.0)
        u_ref[...] = (h0 * w2_ref[0, 0] + h1 * w2_ref[1, 0]) * dis

    return pl.pallas_call(
        body,
        in_specs=[
            pl.BlockSpec(memory_space=pltpu.VMEM),
            pl.BlockSpec(memory_space=pltpu.VMEM),
            pl.BlockSpec(memory_space=pltpu.VMEM),
            pl.BlockSpec(memory_space=pltpu.SMEM),
            pl.BlockSpec(memory_space=pltpu.SMEM),
            pl.BlockSpec(memory_space=pltpu.SMEM),
        ],
        out_shape=jax.ShapeDtypeStruct((1, _NP), _f32),
    )(a4, yst, dis, W1, b1, W2)


def _stage_out(a2, u, dis, b2):
    def body(a_ref, u_ref, dis_ref, b_ref, out_ref):
        a = a_ref[...]
        out_ref[...] = (a[0:1] + a[1:2] - u_ref[...]) * dis_ref[...] + b_ref[0]

    return pl.pallas_call(
        body,
        in_specs=[
            pl.BlockSpec(memory_space=pltpu.VMEM),
            pl.BlockSpec(memory_space=pltpu.VMEM),
            pl.BlockSpec(memory_space=pltpu.VMEM),
            pl.BlockSpec(memory_space=pltpu.SMEM),
        ],
        out_shape=jax.ShapeDtypeStruct((1, _NP), _f32),
    )(a2, u, dis, b2)


def kernel(x, edge_index, W1, b1, W2, b2):
    pad_idx = jnp.full((_EPAD - _E,), _NP - 1, jnp.int32)
    src = jnp.concatenate([edge_index[0].astype(jnp.int32), pad_idx])
    dst = jnp.concatenate([edge_index[1].astype(jnp.int32), pad_idx])
    xt = jnp.pad(x, ((0, _NP - _N), (0, 0))).T

    degp = _sc_degree(dst).reshape(_NC, _NP)
    yst, dis = _stage_norm(degp, xt)

    o0, o1 = _sc_aggregate2(yst[0], yst[1], src, dst)
    a4 = jnp.concatenate([o0.reshape(_NC, _NP), o1.reshape(_NC, _NP)])
    u = _stage_layer1(a4, yst, dis, W1, b1, W2)

    o = _sc_aggregate1(u[0], src, dst)
    orow = _stage_out(o.reshape(_NC, _NP), u, dis, b2)

    return orow[0, :_N].reshape(_N, 1)


# speedup vs baseline: 296.8850x; 2.2330x over previous
import functools

import jax
import jax.numpy as jnp
from jax import lax
from jax.experimental import pallas as pl
from jax.experimental.pallas import tpu as pltpu
from jax.experimental.pallas import tpu_sc as plsc

_N = 100000
_NP = 100096
_E = 6400000
_NC = 2
_NS = 16
_NW = _NC * _NS
_NPT = _NP // _NS
_ET = _E // _NW
_CH = 4000
_SUP = _ET // (2 * _CH)

_f32 = jnp.float32
_mesh = plsc.VectorSubcoreMesh(core_axis_name="c", subcore_axis_name="s",
                               num_cores=_NC, num_subcores=_NS)
_sc_params = pltpu.CompilerParams(use_tc_tiling_on_sc=False)


@functools.partial(
    pl.kernel,
    out_type=jax.ShapeDtypeStruct((_NC * _NP,), _f32),
    mesh=_mesh,
    compiler_params=_sc_params,
    scratch_types=[
        pltpu.VMEM_SHARED((_NP,), _f32),
        pltpu.VMEM((_NPT,), _f32),
        pltpu.VMEM((_CH,), jnp.int32),
        pltpu.VMEM((_CH,), jnp.int32),
        pltpu.SemaphoreType.DMA,
        pltpu.SemaphoreType.DMA,
    ],
)
def _sc_degree(ei_hbm, out_hbm, acc_sh, ones_v, dst_a, dst_b, sem_a, sem_b):
    c = lax.axis_index("c")
    s = lax.axis_index("s")
    wid = c * _NS + s

    @pl.loop(0, _NPT // 16)
    def _fill(i):
        ones_v[pl.ds(i * 16, 16)] = jnp.ones((16,), _f32)

    nb = s * _NPT
    pltpu.sync_copy(ones_v, acc_sh.at[pl.ds(nb, _NPT)])
    plsc.subcore_barrier()

    e0 = wid * _ET

    @pl.loop(0, _SUP)
    def _sup(t):
        eb = e0 + t * (2 * _CH)
        pltpu.sync_copy(ei_hbm.at[1, pl.ds(eb, _CH)], dst_a)
        da = pltpu.async_copy(ones_v.at[pl.ds(0, _CH)], acc_sh.at[dst_a],
                              sem_a, add=True)
        pltpu.sync_copy(ei_hbm.at[1, pl.ds(eb + _CH, _CH)], dst_b)
        db = pltpu.async_copy(ones_v.at[pl.ds(0, _CH)], acc_sh.at[dst_b],
                              sem_b, add=True)
        da.wait()
        db.wait()

    plsc.subcore_barrier()
    pltpu.sync_copy(acc_sh.at[pl.ds(nb, _NPT)], ones_v)
    pltpu.sync_copy(ones_v, out_hbm.at[pl.ds(c * _NP + nb, _NPT)])


@functools.partial(
    pl.kernel,
    out_type=(jax.ShapeDtypeStruct((_NC * _NP,), _f32),
              jax.ShapeDtypeStruct((_NC * _NP,), _f32)),
    mesh=_mesh,
    compiler_params=_sc_params,
    scratch_types=[
        pltpu.VMEM_SHARED((_NP,), _f32),
        pltpu.VMEM_SHARED((_NP,), _f32),
        pltpu.VMEM_SHARED((_NP,), _f32),
        pltpu.VMEM_SHARED((_NP,), _f32),
        pltpu.VMEM((_NPT,), _f32),
        pltpu.VMEM((_CH,), jnp.int32),
        pltpu.VMEM((_CH,), jnp.int32),
        pltpu.VMEM((_CH,), jnp.int32),
        pltpu.VMEM((_CH,), jnp.int32),
        pltpu.VMEM((_CH,), _f32),
        pltpu.VMEM((_CH,), _f32),
        pltpu.VMEM((_CH,), _f32),
        pltpu.VMEM((_CH,), _f32),
        pltpu.SemaphoreType.DMA,
        pltpu.SemaphoreType.DMA,
        pltpu.SemaphoreType.DMA,
        pltpu.SemaphoreType.DMA,
        pltpu.SemaphoreType.DMA,
        pltpu.SemaphoreType.DMA,
    ],
)
def _sc_aggregate2(y0_hbm, y1_hbm, ei_hbm, o0_hbm, o1_hbm,
                   t0_sh, t1_sh, a0_sh, a1_sh, stage_v,
                   src_a, dst_a, src_b, dst_b, m0a, m1a, m0b, m1b,
                   g0sem, g1sem, s0a_sem, s1a_sem, s0b_sem, s1b_sem):
    c = lax.axis_index("c")
    s = lax.axis_index("s")
    wid = c * _NS + s
    nb = s * _NPT

    pltpu.sync_copy(y0_hbm.at[pl.ds(nb, _NPT)], stage_v)
    pltpu.sync_copy(stage_v, t0_sh.at[pl.ds(nb, _NPT)])
    pltpu.sync_copy(stage_v, a0_sh.at[pl.ds(nb, _NPT)])
    pltpu.sync_copy(y1_hbm.at[pl.ds(nb, _NPT)], stage_v)
    pltpu.sync_copy(stage_v, t1_sh.at[pl.ds(nb, _NPT)])
    pltpu.sync_copy(stage_v, a1_sh.at[pl.ds(nb, _NPT)])
    plsc.subcore_barrier()

    e0 = wid * _ET

    @pl.loop(0, _SUP)
    def _sup(t):
        eb = e0 + t * (2 * _CH)
        pltpu.sync_copy(ei_hbm.at[0, pl.ds(eb, _CH)], src_a)
        pltpu.sync_copy(ei_hbm.at[1, pl.ds(eb, _CH)], dst_a)
        g0 = pltpu.async_copy(t0_sh.at[src_a], m0a, g0sem)
        g1 = pltpu.async_copy(t1_sh.at[src_a], m1a, g1sem)
        pltpu.sync_copy(ei_hbm.at[0, pl.ds(eb + _CH, _CH)], src_b)
        pltpu.sync_copy(ei_hbm.at[1, pl.ds(eb + _CH, _CH)], dst_b)
        g0.wait()
        g1.wait()
        s0a = pltpu.async_copy(m0a, a0_sh.at[dst_a], s0a_sem, add=True)
        s1a = pltpu.async_copy(m1a, a1_sh.at[dst_a], s1a_sem, add=True)
        g0b = pltpu.async_copy(t0_sh.at[src_b], m0b, g0sem)
        g1b = pltpu.async_copy(t1_sh.at[src_b], m1b, g1sem)
        g0b.wait()
        g1b.wait()
        s0b = pltpu.async_copy(m0b, a0_sh.at[dst_b], s0b_sem, add=True)
        s1b = pltpu.async_copy(m1b, a1_sh.at[dst_b], s1b_sem, add=True)
        s0a.wait()
        s1a.wait()
        s0b.wait()
        s1b.wait()

    plsc.subcore_barrier()
    pltpu.sync_copy(a0_sh.at[pl.ds(nb, _NPT)], stage_v)
    pltpu.sync_copy(stage_v, o0_hbm.at[pl.ds(c * _NP + nb, _NPT)])
    pltpu.sync_copy(a1_sh.at[pl.ds(nb, _NPT)], stage_v)
    pltpu.sync_copy(stage_v, o1_hbm.at[pl.ds(c * _NP + nb, _NPT)])


@functools.partial(
    pl.kernel,
    out_type=jax.ShapeDtypeStruct((_NC * _NP,), _f32),
    mesh=_mesh,
    compiler_params=_sc_params,
    scratch_types=[
        pltpu.VMEM_SHARED((_NP,), _f32),
        pltpu.VMEM_SHARED((_NP,), _f32),
        pltpu.VMEM((_NPT,), _f32),
        pltpu.VMEM((_CH,), jnp.int32),
        pltpu.VMEM((_CH,), jnp.int32),
        pltpu.VMEM((_CH,), jnp.int32),
        pltpu.VMEM((_CH,), jnp.int32),
        pltpu.VMEM((_CH,), _f32),
        pltpu.VMEM((_CH,), _f32),
        pltpu.SemaphoreType.DMA,
        pltpu.SemaphoreType.DMA,
        pltpu.SemaphoreType.DMA,
    ],
)
def _sc_aggregate1(u_hbm, ei_hbm, o_hbm,
                   t_sh, a_sh, stage_v, src_a, dst_a, src_b, dst_b,
                   m_a, m_b, gsem, sa_sem, sb_sem):
    c = lax.axis_index("c")
    s = lax.axis_index("s")
    wid = c * _NS + s
    nb = s * _NPT

    pltpu.sync_copy(u_hbm.at[pl.ds(nb, _NPT)], stage_v)
    pltpu.sync_copy(stage_v, t_sh.at[pl.ds(nb, _NPT)])
    pltpu.sync_copy(stage_v, a_sh.at[pl.ds(nb, _NPT)])
    plsc.subcore_barrier()

    e0 = wid * _ET

    @pl.loop(0, _SUP)
    def _sup(t):
        eb = e0 + t * (2 * _CH)
        pltpu.sync_copy(ei_hbm.at[0, pl.ds(eb, _CH)], src_a)
        pltpu.sync_copy(ei_hbm.at[1, pl.ds(eb, _CH)], dst_a)
        g = pltpu.async_copy(t_sh.at[src_a], m_a, gsem)
        pltpu.sync_copy(ei_hbm.at[0, pl.ds(eb + _CH, _CH)], src_b)
        pltpu.sync_copy(ei_hbm.at[1, pl.ds(eb + _CH, _CH)], dst_b)
        g.wait()
        sa = pltpu.async_copy(m_a, a_sh.at[dst_a], sa_sem, add=True)
        gb = pltpu.async_copy(t_sh.at[src_b], m_b, gsem)
        gb.wait()
        sb = pltpu.async_copy(m_b, a_sh.at[dst_b], sb_sem, add=True)
        sa.wait()
        sb.wait()

    plsc.subcore_barrier()
    pltpu.sync_copy(a_sh.at[pl.ds(nb, _NPT)], stage_v)
    pltpu.sync_copy(stage_v, o_hbm.at[pl.ds(c * _NP + nb, _NPT)])


def _stage_norm(degp, xt):
    def body(degp_ref, xt_ref, yst_ref, dis_ref):
        deg = degp_ref[0:1, :] + degp_ref[1:2, :] - 1.0
        dis = lax.rsqrt(deg)
        dis_ref[...] = dis
        yst_ref[...] = xt_ref[...] * dis

    return pl.pallas_call(
        body,
        out_shape=(
            jax.ShapeDtypeStruct((2, _NP), _f32),
            jax.ShapeDtypeStruct((1, _NP), _f32),
        ),
    )(degp, xt)


def _stage_layer1(a4, yst, dis, W1, b1, W2):
    def body(a_ref, yst_ref, dis_ref, w1_ref, b1_ref, w2_ref, u_ref):
        a = a_ref[...]
        ys = yst_ref[...]
        dis = dis_ref[...]
        z0 = (a[0:1] + a[1:2] - ys[0:1]) * dis
        z1 = (a[2:3] + a[3:4] - ys[1:2]) * dis
        h0 = jnp.maximum(z0 * w1_ref[0, 0] + z1 * w1_ref[1, 0] + b1_ref[0],
                         0.0)
        h1 = jnp.maximum(z0 * w1_ref[0, 1] + z1 * w1_ref[1, 1] + b1_ref[1],
                         0.0)
        u_ref[...] = (h0 * w2_ref[0, 0] + h1 * w2_ref[1, 0]) * dis

    return pl.pallas_call(
        body,
        in_specs=[
            pl.BlockSpec(memory_space=pltpu.VMEM),
            pl.BlockSpec(memory_space=pltpu.VMEM),
            pl.BlockSpec(memory_space=pltpu.VMEM),
            pl.BlockSpec(memory_space=pltpu.SMEM),
            pl.BlockSpec(memory_space=pltpu.SMEM),
            pl.BlockSpec(memory_space=pltpu.SMEM),
        ],
        out_shape=jax.ShapeDtypeStruct((1, _NP), _f32),
    )(a4, yst, dis, W1, b1, W2)


def _stage_out(a2, u, dis, b2):
    def body(a_ref, u_ref, dis_ref, b_ref, out_ref):
        a = a_ref[...]
        out_ref[...] = (a[0:1] + a[1:2] - u_ref[...]) * dis_ref[...] + b_ref[0]

    return pl.pallas_call(
        body,
        in_specs=[
            pl.BlockSpec(memory_space=pltpu.VMEM),
            pl.BlockSpec(memory_space=pltpu.VMEM),
            pl.BlockSpec(memory_space=pltpu.VMEM),
            pl.BlockSpec(memory_space=pltpu.SMEM),
        ],
        out_shape=jax.ShapeDtypeStruct((1, _NP), _f32),
    )(a2, u, dis, b2)


def kernel(x, edge_index, W1, b1, W2, b2):
    ei = edge_index.astype(jnp.int32)
    xt = jnp.pad(x, ((0, _NP - _N), (0, 0))).T

    degp = _sc_degree(ei).reshape(_NC, _NP)
    yst, dis = _stage_norm(degp, xt)

    o0, o1 = _sc_aggregate2(yst[0], yst[1], ei)
    a4 = jnp.concatenate([o0.reshape(_NC, _NP), o1.reshape(_NC, _NP)])
    u = _stage_layer1(a4, yst, dis, W1, b1, W2)

    o = _sc_aggregate1(u[0], ei)
    orow = _stage_out(o.reshape(_NC, _NP), u, dis, b2)

    return orow[0, :_N].reshape(_N, 1)


# speedup vs baseline: 335.0463x; 1.1285x over previous
import functools

import jax
import jax.numpy as jnp
from jax import lax
from jax.experimental import pallas as pl
from jax.experimental.pallas import tpu as pltpu
from jax.experimental.pallas import tpu_sc as plsc

_N = 100000
_NP = 100096
_E = 6400000
_NC = 2
_NS = 16
_NW = _NC * _NS
_NPT = _NP // _NS
_ET = _E // _NW
_CH = 4000
_SUP = _ET // (2 * _CH)

_f32 = jnp.float32
_mesh = plsc.VectorSubcoreMesh(core_axis_name="c", subcore_axis_name="s",
                               num_cores=_NC, num_subcores=_NS)
_sc_params = pltpu.CompilerParams(use_tc_tiling_on_sc=False)


@functools.partial(
    pl.kernel,
    out_type=jax.ShapeDtypeStruct((_NC * _NP,), _f32),
    mesh=_mesh,
    compiler_params=_sc_params,
    scratch_types=[
        pltpu.VMEM_SHARED((_NP,), _f32),
        pltpu.VMEM((_NPT,), _f32),
        pltpu.VMEM((_CH,), jnp.int32),
        pltpu.VMEM((_CH,), jnp.int32),
        pltpu.SemaphoreType.DMA,
        pltpu.SemaphoreType.DMA,
    ],
)
def _sc_degree(ei_hbm, out_hbm, acc_sh, ones_v, dst_a, dst_b, sem_a, sem_b):
    c = lax.axis_index("c")
    s = lax.axis_index("s")
    wid = c * _NS + s

    @pl.loop(0, _NPT // 16)
    def _fill(i):
        ones_v[pl.ds(i * 16, 16)] = jnp.ones((16,), _f32)

    nb = s * _NPT
    pltpu.sync_copy(ones_v, acc_sh.at[pl.ds(nb, _NPT)])
    plsc.subcore_barrier()

    e0 = wid * _ET

    @pl.loop(0, _SUP)
    def _sup(t):
        eb = e0 + t * (2 * _CH)
        pltpu.sync_copy(ei_hbm.at[1, pl.ds(eb, _CH)], dst_a)
        da = pltpu.async_copy(ones_v.at[pl.ds(0, _CH)], acc_sh.at[dst_a],
                              sem_a, add=True)
        pltpu.sync_copy(ei_hbm.at[1, pl.ds(eb + _CH, _CH)], dst_b)
        db = pltpu.async_copy(ones_v.at[pl.ds(0, _CH)], acc_sh.at[dst_b],
                              sem_b, add=True)
        da.wait()
        db.wait()

    plsc.subcore_barrier()
    pltpu.sync_copy(acc_sh.at[pl.ds(nb, _NPT)], ones_v)
    pltpu.sync_copy(ones_v, out_hbm.at[pl.ds(c * _NP + nb, _NPT)])


@functools.partial(
    pl.kernel,
    out_type=(jax.ShapeDtypeStruct((_NC * _NP,), _f32),
              jax.ShapeDtypeStruct((_NC * _NP,), _f32)),
    mesh=_mesh,
    compiler_params=_sc_params,
    scratch_types=[
        pltpu.VMEM_SHARED((_NP,), jnp.int32),
        pltpu.VMEM_SHARED((_NP,), _f32),
        pltpu.VMEM_SHARED((_NP,), _f32),
        pltpu.VMEM((_NPT,), _f32),
        pltpu.VMEM((_NPT,), jnp.int32),
        pltpu.VMEM((_CH,), jnp.int32),
        pltpu.VMEM((_CH,), jnp.int32),
        pltpu.VMEM((_CH,), jnp.int32),
        pltpu.VMEM((_CH,), jnp.int32),
        pltpu.VMEM((_CH,), jnp.int32),
        pltpu.VMEM((_CH,), jnp.int32),
        pltpu.VMEM((_CH,), _f32),
        pltpu.VMEM((_CH,), _f32),
        pltpu.VMEM((_CH,), _f32),
        pltpu.VMEM((_CH,), _f32),
        pltpu.SemaphoreType.DMA,
        pltpu.SemaphoreType.DMA,
        pltpu.SemaphoreType.DMA,
        pltpu.SemaphoreType.DMA,
        pltpu.SemaphoreType.DMA,
    ],
)
def _sc_aggregate2(ypk_hbm, y0_hbm, y1_hbm, ei_hbm, o0_hbm, o1_hbm,
                   t_sh, a0_sh, a1_sh, stage_v, stage_i,
                   src_a, dst_a, src_b, dst_b, mpa, mpb,
                   m0a, m1a, m0b, m1b,
                   gsem, s0a_sem, s1a_sem, s0b_sem, s1b_sem):
    c = lax.axis_index("c")
    s = lax.axis_index("s")
    wid = c * _NS + s
    nb = s * _NPT

    pltpu.sync_copy(ypk_hbm.at[pl.ds(nb, _NPT)], stage_i)
    pltpu.sync_copy(stage_i, t_sh.at[pl.ds(nb, _NPT)])
    pltpu.sync_copy(y0_hbm.at[pl.ds(nb, _NPT)], stage_v)
    pltpu.sync_copy(stage_v, a0_sh.at[pl.ds(nb, _NPT)])
    pltpu.sync_copy(y1_hbm.at[pl.ds(nb, _NPT)], stage_v)
    pltpu.sync_copy(stage_v, a1_sh.at[pl.ds(nb, _NPT)])
    plsc.subcore_barrier()

    e0 = wid * _ET
    himask = jnp.full((16,), -65536, jnp.int32)

    def _unpack(mp, m0, m1):
        @pl.loop(0, _CH // 16)
        def _up(i):
            v = mp[pl.ds(i * 16, 16)]
            m0[pl.ds(i * 16, 16)] = lax.bitcast_convert_type(
                lax.shift_left(v, jnp.full((16,), 16, jnp.int32)), _f32)
            m1[pl.ds(i * 16, 16)] = lax.bitcast_convert_type(
                lax.bitwise_and(v, himask), _f32)

    @pl.loop(0, _SUP)
    def _sup(t):
        eb = e0 + t * (2 * _CH)
        pltpu.sync_copy(ei_hbm.at[0, pl.ds(eb, _CH)], src_a)
        pltpu.sync_copy(ei_hbm.at[1, pl.ds(eb, _CH)], dst_a)
        ga = pltpu.async_copy(t_sh.at[src_a], mpa, gsem)
        pltpu.sync_copy(ei_hbm.at[0, pl.ds(eb + _CH, _CH)], src_b)
        pltpu.sync_copy(ei_hbm.at[1, pl.ds(eb + _CH, _CH)], dst_b)
        ga.wait()
        gb = pltpu.async_copy(t_sh.at[src_b], mpb, gsem)
        _unpack(mpa, m0a, m1a)
        s0a = pltpu.async_copy(m0a, a0_sh.at[dst_a], s0a_sem, add=True)
        s1a = pltpu.async_copy(m1a, a1_sh.at[dst_a], s1a_sem, add=True)
        gb.wait()
        _unpack(mpb, m0b, m1b)
        s0b = pltpu.async_copy(m0b, a0_sh.at[dst_b], s0b_sem, add=True)
        s1b = pltpu.async_copy(m1b, a1_sh.at[dst_b], s1b_sem, add=True)
        s0a.wait()
        s1a.wait()
        s0b.wait()
        s1b.wait()

    plsc.subcore_barrier()
    pltpu.sync_copy(a0_sh.at[pl.ds(nb, _NPT)], stage_v)
    pltpu.sync_copy(stage_v, o0_hbm.at[pl.ds(c * _NP + nb, _NPT)])
    pltpu.sync_copy(a1_sh.at[pl.ds(nb, _NPT)], stage_v)
    pltpu.sync_copy(stage_v, o1_hbm.at[pl.ds(c * _NP + nb, _NPT)])


@functools.partial(
    pl.kernel,
    out_type=jax.ShapeDtypeStruct((_NC * _NP,), _f32),
    mesh=_mesh,
    compiler_params=_sc_params,
    scratch_types=[
        pltpu.VMEM_SHARED((_NP,), _f32),
        pltpu.VMEM_SHARED((_NP,), _f32),
        pltpu.VMEM((_NPT,), _f32),
        pltpu.VMEM((_CH,), jnp.int32),
        pltpu.VMEM((_CH,), jnp.int32),
        pltpu.VMEM((_CH,), jnp.int32),
        pltpu.VMEM((_CH,), jnp.int32),
        pltpu.VMEM((_CH,), _f32),
        pltpu.VMEM((_CH,), _f32),
        pltpu.SemaphoreType.DMA,
        pltpu.SemaphoreType.DMA,
        pltpu.SemaphoreType.DMA,
    ],
)
def _sc_aggregate1(u_hbm, ei_hbm, o_hbm,
                   t_sh, a_sh, stage_v, src_a, dst_a, src_b, dst_b,
                   m_a, m_b, gsem, sa_sem, sb_sem):
    c = lax.axis_index("c")
    s = lax.axis_index("s")
    wid = c * _NS + s
    nb = s * _NPT

    pltpu.sync_copy(u_hbm.at[pl.ds(nb, _NPT)], stage_v)
    pltpu.sync_copy(stage_v, t_sh.at[pl.ds(nb, _NPT)])
    pltpu.sync_copy(stage_v, a_sh.at[pl.ds(nb, _NPT)])
    plsc.subcore_barrier()

    e0 = wid * _ET

    @pl.loop(0, _SUP)
    def _sup(t):
        eb = e0 + t * (2 * _CH)
        pltpu.sync_copy(ei_hbm.at[0, pl.ds(eb, _CH)], src_a)
        pltpu.sync_copy(ei_hbm.at[1, pl.ds(eb, _CH)], dst_a)
        g = pltpu.async_copy(t_sh.at[src_a], m_a, gsem)
        pltpu.sync_copy(ei_hbm.at[0, pl.ds(eb + _CH, _CH)], src_b)
        pltpu.sync_copy(ei_hbm.at[1, pl.ds(eb + _CH, _CH)], dst_b)
        g.wait()
        sa = pltpu.async_copy(m_a, a_sh.at[dst_a], sa_sem, add=True)
        gb = pltpu.async_copy(t_sh.at[src_b], m_b, gsem)
        gb.wait()
        sb = pltpu.async_copy(m_b, a_sh.at[dst_b], sb_sem, add=True)
        sa.wait()
        sb.wait()

    plsc.subcore_barrier()
    pltpu.sync_copy(a_sh.at[pl.ds(nb, _NPT)], stage_v)
    pltpu.sync_copy(stage_v, o_hbm.at[pl.ds(c * _NP + nb, _NPT)])


def _stage_norm(degp, xt):
    def body(degp_ref, xt_ref, yst_ref, dis_ref, ypk_ref):
        deg = degp_ref[0:1, :] + degp_ref[1:2, :] - 1.0
        dis = lax.rsqrt(deg)
        dis_ref[...] = dis
        ys = xt_ref[...] * dis
        yst_ref[...] = ys
        b0 = lax.bitcast_convert_type(ys[0:1], jnp.int32) + 0x8000
        b1 = lax.bitcast_convert_type(ys[1:2], jnp.int32) + 0x8000
        ypk_ref[...] = jnp.bitwise_or(
            jnp.bitwise_and(b1, jnp.int32(-65536)),
            lax.shift_right_logical(b0, 16),
        )

    return pl.pallas_call(
        body,
        out_shape=(
            jax.ShapeDtypeStruct((2, _NP), _f32),
            jax.ShapeDtypeStruct((1, _NP), _f32),
            jax.ShapeDtypeStruct((1, _NP), jnp.int32),
        ),
    )(degp, xt)


def _stage_layer1(a4, yst, dis, W1, b1, W2):
    def body(a_ref, yst_ref, dis_ref, w1_ref, b1_ref, w2_ref, u_ref):
        a = a_ref[...]
        ys = yst_ref[...]
        dis = dis_ref[...]
        z0 = (a[0:1] + a[1:2] - ys[0:1]) * dis
        z1 = (a[2:3] + a[3:4] - ys[1:2]) * dis
        h0 = jnp.maximum(z0 * w1_ref[0, 0] + z1 * w1_ref[1, 0] + b1_ref[0],
                         0.0)
        h1 = jnp.maximum(z0 * w1_ref[0, 1] + z1 * w1_ref[1, 1] + b1_ref[1],
                         0.0)
        u_ref[...] = (h0 * w2_ref[0, 0] + h1 * w2_ref[1, 0]) * dis

    return pl.pallas_call(
        body,
        in_specs=[
            pl.BlockSpec(memory_space=pltpu.VMEM),
            pl.BlockSpec(memory_space=pltpu.VMEM),
            pl.BlockSpec(memory_space=pltpu.VMEM),
            pl.BlockSpec(memory_space=pltpu.SMEM),
            pl.BlockSpec(memory_space=pltpu.SMEM),
            pl.BlockSpec(memory_space=pltpu.SMEM),
        ],
        out_shape=jax.ShapeDtypeStruct((1, _NP), _f32),
    )(a4, yst, dis, W1, b1, W2)


def _stage_out(a2, u, dis, b2):
    def body(a_ref, u_ref, dis_ref, b_ref, out_ref):
        a = a_ref[...]
        out_ref[...] = (a[0:1] + a[1:2] - u_ref[...]) * dis_ref[...] + b_ref[0]

    return pl.pallas_call(
        body,
        in_specs=[
            pl.BlockSpec(memory_space=pltpu.VMEM),
            pl.BlockSpec(memory_space=pltpu.VMEM),
            pl.BlockSpec(memory_space=pltpu.VMEM),
            pl.BlockSpec(memory_space=pltpu.SMEM),
        ],
        out_shape=jax.ShapeDtypeStruct((1, _NP), _f32),
    )(a2, u, dis, b2)


def kernel(x, edge_index, W1, b1, W2, b2):
    ei = edge_index.astype(jnp.int32)
    xt = jnp.pad(x, ((0, _NP - _N), (0, 0))).T

    degp = _sc_degree(ei).reshape(_NC, _NP)
    yst, dis, ypk = _stage_norm(degp, xt)

    o0, o1 = _sc_aggregate2(ypk[0], yst[0], yst[1], ei)
    a4 = jnp.concatenate([o0.reshape(_NC, _NP), o1.reshape(_NC, _NP)])
    u = _stage_layer1(a4, yst, dis, W1, b1, W2)

    o = _sc_aggregate1(u[0], ei)
    orow = _stage_out(o.reshape(_NC, _NP), u, dis, b2)

    return orow[0, :_N].reshape(_N, 1)
